# Initial kernel scaffold; baseline (speedup 1.0000x reference)
import jax
import jax.numpy as jnp
from jax.experimental import pallas as pl


def kernel(feat, edge_index, W_neigh, b_neigh):
    raise NotImplementedError("write your pallas kernel here")


# speedup vs baseline: 6.6960x; 6.6960x over previous
import functools

import jax
import jax.numpy as jnp
from jax import lax
from jax.experimental import pallas as pl
from jax.experimental.pallas import tpu as pltpu
from jax.experimental.pallas import tpu_sc as plsc

NC = 2
NS = 16
GB = 128


@functools.lru_cache(maxsize=None)
def _build_sc_agg(n, e, dp):
    assert e % GB == 0 and n % (8 * NS) == 0
    groups = e // GB
    gmax = (groups + NC * NS - 1) // (NC * NS)
    rpt = n // NS
    mesh = plsc.VectorSubcoreMesh(core_axis_name="c", subcore_axis_name="s")

    @functools.partial(
        pl.kernel,
        mesh=mesh,
        compiler_params=pltpu.CompilerParams(use_tc_tiling_on_sc=False),
        out_type=jax.ShapeDtypeStruct((NC, n, dp), jnp.float32),
        scratch_types=[
            pltpu.VMEM((1, GB), jnp.int32),
            pltpu.VMEM((1, GB), jnp.int32),
            pltpu.VMEM((GB, dp), jnp.float32),
            pltpu.VMEM_SHARED((n, dp), jnp.float32),
            pltpu.SemaphoreType.DMA,
        ],
    )
    def sc_agg(feat_hbm, src_hbm, dst_hbm, zero_hbm, out_hbm,
               sidx, didx, rows, acc, sem):
        c = lax.axis_index("c")
        s = lax.axis_index("s")
        wid = s * NC + c
        pltpu.sync_copy(zero_hbm.at[pl.ds(s * rpt, rpt)],
                        acc.at[pl.ds(s * rpt, rpt)])
        plsc.subcore_barrier()

        def body(k, carry):
            g = wid + k * (NC * NS)

            @pl.when(g < groups)
            def _():
                pltpu.sync_copy(src_hbm.at[pl.ds(g * GB, GB)], sidx.at[0])
                pltpu.sync_copy(dst_hbm.at[pl.ds(g * GB, GB)], didx.at[0])
                pltpu.async_copy(feat_hbm.at[sidx.at[0]], rows, sem).wait()
                pltpu.sync_copy(rows, acc.at[didx.at[0]], add=True)

            return carry

        lax.fori_loop(0, gmax, body, 0)
        plsc.subcore_barrier()
        pltpu.sync_copy(acc.at[pl.ds(s * rpt, rpt)],
                        out_hbm.at[c, pl.ds(s * rpt, rpt)])

    return sc_agg


def _tc_body(p_ref, feat_ref, w_ref, b_ref, out_ref):
    acc = p_ref[0] + p_ref[1]
    agg = acc[:, :128]
    deg = jnp.sum(acc[:, 128:144], axis=1, keepdims=True)
    h = (agg + feat_ref[...]) / (deg + 1.0)
    r = lax.dot_general(h, w_ref[...], (((1,), (1,)), ((), ())),
                        preferred_element_type=jnp.float32)
    r = r + b_ref[...]
    out_ref[...] = jnp.where(r >= 0, r, 0.01 * r)


def kernel(feat, edge_index, W_neigh, b_neigh):
    n, d = feat.shape
    e = edge_index.shape[1]
    dp = d + 16
    npad = -(-n // 128) * 128
    src = edge_index[0]
    dst = edge_index[1]
    feat_pad = jnp.concatenate(
        [feat,
         jnp.ones((n, 1), jnp.float32),
         jnp.zeros((n, 15), jnp.float32)], axis=1)
    zero_init = jnp.zeros((npad, dp), jnp.float32)

    partials = _build_sc_agg(npad, e, dp)(feat_pad, src, dst, zero_init)

    bn = 1000 if n % 1000 == 0 else n
    grid = n // bn
    out = pl.pallas_call(
        _tc_body,
        grid=(grid,),
        in_specs=[
            pl.BlockSpec((NC, bn, dp), lambda i: (0, i, 0)),
            pl.BlockSpec((bn, d), lambda i: (i, 0)),
            pl.BlockSpec(W_neigh.shape, lambda i: (0, 0)),
            pl.BlockSpec((1, b_neigh.shape[0]), lambda i: (0, 0)),
        ],
        out_specs=pl.BlockSpec((bn, d), lambda i: (i, 0)),
        out_shape=jax.ShapeDtypeStruct((n, d), jnp.float32),
    )(partials, feat, W_neigh, b_neigh.reshape(1, -1))
    return out
